# baseline (speedup 1.0000x reference)
import jax
import jax.numpy as jnp
from jax import lax
from jax.experimental import pallas as pl
from jax.experimental.pallas import tpu as pltpu
from jax.experimental.pallas import tpu_sc as plsc

N, E, D, H, DE, B = 10000, 160000, 256, 512, 16, 64
DH = D // 2
NB = 10
BN_ROWS = N // NB
CH = 64
NSC, NSUB = 2, 16
NCHUNKS = E // CH
CPS = NCHUNKS // NSUB
REM = NCHUNKS - CPS * NSUB
EB = 3200
EB8 = EB // 8
NEB = E // EB


def _f32(x):
    return x.astype(jnp.float32)


def _h_body(x_ref, b_ref, vn_ref, h0_ref, h1_ref):
    b = b_ref[0, 0, :]
    oh = (b[:, None] == lax.broadcasted_iota(jnp.int32, (BN_ROWS, B), 1))
    h = x_ref[...] + jnp.dot(_f32(oh), vn_ref[...],
                             preferred_element_type=jnp.float32,
                             precision=lax.Precision.HIGHEST)
    h0_ref[...] = h[:, :DH]
    h1_ref[...] = h[:, DH:]


def _h_halves(x, batch3, vn):
    return pl.pallas_call(
        _h_body,
        grid=(NB,),
        in_specs=[
            pl.BlockSpec((BN_ROWS, D), lambda i: (i, 0)),
            pl.BlockSpec((1, 1, BN_ROWS), lambda i: (i, 0, 0)),
            pl.BlockSpec((B, D), lambda i: (0, 0)),
        ],
        out_specs=[
            pl.BlockSpec((BN_ROWS, DH), lambda i: (i, 0)),
            pl.BlockSpec((BN_ROWS, DH), lambda i: (i, 0)),
        ],
        out_shape=[
            jax.ShapeDtypeStruct((N, DH), jnp.float32),
            jax.ShapeDtypeStruct((N, DH), jnp.float32),
        ],
        compiler_params=pltpu.CompilerParams(
            dimension_semantics=("parallel",)),
    )(x, batch3, vn)


def _ep_body(ea_ref, W8_ref, eb8_ref, ep0_ref, ep1_ref):
    out8 = jnp.dot(ea_ref[...], W8_ref[...],
                   preferred_element_type=jnp.float32) + eb8_ref[...]
    for j in range(8):
        ep0_ref[:, j, :] = out8[:, j * D:j * D + DH]
        ep1_ref[:, j, :] = out8[:, j * D + DH:(j + 1) * D]


def _ep_halves(ea8, W8, eb8):
    return pl.pallas_call(
        _ep_body,
        grid=(NEB,),
        in_specs=[
            pl.BlockSpec((EB8, 8 * DE), lambda i: (i, 0)),
            pl.BlockSpec((8 * DE, 8 * D), lambda i: (0, 0)),
            pl.BlockSpec((1, 8 * D), lambda i: (0, 0)),
        ],
        out_specs=[
            pl.BlockSpec((EB8, 8, DH), lambda i: (i, 0, 0)),
            pl.BlockSpec((EB8, 8, DH), lambda i: (i, 0, 0)),
        ],
        out_shape=[
            jax.ShapeDtypeStruct((E // 8, 8, DH), jnp.float32),
            jax.ShapeDtypeStruct((E // 8, 8, DH), jnp.float32),
        ],
        compiler_params=pltpu.CompilerParams(
            dimension_semantics=("parallel",)),
    )(ea8, W8, eb8)


def _sc_body(h0, h1, ep0, ep1, eidx, zp0, zp1,
             idx0, idx1, idx2, rows0, rows1, rows2, epv0, epv1, epv2,
             upd, sem_g, sem_e, sem_s):
    c = lax.axis_index("c")
    s = lax.axis_index("s")
    idxs = (idx0, idx1, idx2)
    rowss = (rows0, rows1, rows2)
    epvs = (epv0, epv1, epv2)
    RCH = 400
    NRCH = N // RCH

    def run_half(h_ref, ep_ref, zp_ref):
        @pl.loop(s, NRCH, step=NSUB)
        def _(rc):
            pltpu.sync_copy(h_ref.at[pl.ds(rc * RCH, RCH)],
                            upd.at[pl.ds(rc * RCH, RCH)])
        plsc.subcore_barrier()

        base = s * CPS

        def fire(slot, k):
            pltpu.sync_copy(eidx.at[k], idxs[slot])
            pltpu.async_copy(h_ref.at[idxs[slot].at[0]], rowss[slot],
                             sem_g.at[slot])
            pltpu.async_copy(ep_ref.at[pl.ds(k * (CH // 8), CH // 8)],
                             epvs[slot], sem_e.at[slot])

        def wait_ge(slot):
            pltpu.make_async_copy(h_ref.at[idxs[slot].at[0]], rowss[slot],
                                  sem_g.at[slot]).wait()
            pltpu.make_async_copy(ep_ref.at[pl.ds(0, CH // 8)], epvs[slot],
                                  sem_e.at[slot]).wait()

        def compute(slot):
            rows, epv = rowss[slot], epvs[slot]

            @plsc.parallel_loop(0, CH // 8)
            def _(rr):
                for jj in range(8):
                    for j in range(DH // 16):
                        sl = pl.ds(j * 16, 16)
                        rows[rr * 8 + jj, sl] = jnp.maximum(
                            rows[rr * 8 + jj, sl] + epv[rr, jj, sl], 0.0)

        def fire_scatter(slot):
            pltpu.async_copy(rowss[slot], upd.at[idxs[slot].at[1]],
                             sem_s.at[slot], add=True)

        def wait_scatter(slot):
            pltpu.make_async_copy(rowss[slot], upd.at[idxs[slot].at[1]],
                                  sem_s.at[slot]).wait()

        fire(0, base)
        fire(1, base + 1)

        @pl.loop(0, CPS // 3)
        def _(q):
            kk = 3 * q

            wait_ge(0)

            @pl.when(q > 0)
            def _():
                wait_scatter(2)
            fire(2, base + kk + 2)
            compute(0)
            fire_scatter(0)

            wait_ge(1)
            wait_scatter(0)

            @pl.when(q < CPS // 3 - 1)
            def _():
                fire(0, base + kk + 3)
            compute(1)
            fire_scatter(1)

            wait_ge(2)
            wait_scatter(1)

            @pl.when(q < CPS // 3 - 1)
            def _():
                fire(1, base + kk + 4)
            compute(2)
            fire_scatter(2)

        wait_scatter(2)

        @pl.when(s < REM)
        def _():
            k = NSUB * CPS + s
            pltpu.sync_copy(eidx.at[k], idx0)
            pltpu.async_copy(h_ref.at[idx0.at[0]], rows0, sem_g.at[0]).wait()
            pltpu.sync_copy(ep_ref.at[pl.ds(k * (CH // 8), CH // 8)], epv0)
            compute(0)
            pltpu.sync_copy(rows0, upd.at[idx0.at[1]], add=True)

        plsc.subcore_barrier()

        @pl.loop(s, NRCH, step=NSUB)
        def _(rc):
            pltpu.sync_copy(upd.at[pl.ds(rc * RCH, RCH)],
                            zp_ref.at[pl.ds(rc * RCH, RCH)])

    @pl.when(c == 0)
    def _():
        run_half(h0, ep0, zp0)

    @pl.when(c == 1)
    def _():
        run_half(h1, ep1, zp1)


def _sc_scatter(h0, h1, ep0, ep1, eidx):
    mesh = plsc.VectorSubcoreMesh(core_axis_name="c", subcore_axis_name="s",
                                  num_cores=NSC, num_subcores=NSUB)
    k = pl.kernel(
        _sc_body,
        out_type=[
            jax.ShapeDtypeStruct((N, DH), jnp.float32),
            jax.ShapeDtypeStruct((N, DH), jnp.float32),
        ],
        mesh=mesh,
        scratch_types=[
            pltpu.VMEM((2, CH), jnp.int32),
            pltpu.VMEM((2, CH), jnp.int32),
            pltpu.VMEM((2, CH), jnp.int32),
            pltpu.VMEM((CH, DH), jnp.float32),
            pltpu.VMEM((CH, DH), jnp.float32),
            pltpu.VMEM((CH, DH), jnp.float32),
            pltpu.VMEM((CH // 8, 8, DH), jnp.float32),
            pltpu.VMEM((CH // 8, 8, DH), jnp.float32),
            pltpu.VMEM((CH // 8, 8, DH), jnp.float32),
            pltpu.VMEM_SHARED((N, DH), jnp.float32),
            pltpu.SemaphoreType.DMA((3,)),
            pltpu.SemaphoreType.DMA((3,)),
            pltpu.SemaphoreType.DMA((3,)),
        ],
    )
    return k(h0, h1, ep0, ep1, eidx)


def _mlp_body(zp0_ref, zp1_ref, W1_ref, b1_ref, W2_ref, b2_ref,
              z2_ref, S_ref):
    zc = jnp.concatenate([zp0_ref[...], zp1_ref[...]], axis=1)
    t = jnp.maximum(jnp.dot(zc, W1_ref[...],
                            preferred_element_type=jnp.float32)
                    + b1_ref[...], 0.0)
    z2 = jnp.maximum(jnp.dot(t, W2_ref[...],
                             preferred_element_type=jnp.float32)
                     + b2_ref[...], 0.0)
    z2_ref[...] = z2
    S_ref[0, 0, :] = jnp.sum(z2, axis=0)
    S_ref[0, 1, :] = jnp.sum(z2 * z2, axis=0)


def _mlp(zp0, zp1, W1, b12, W2, b22):
    return pl.pallas_call(
        _mlp_body,
        grid=(NB,),
        in_specs=[
            pl.BlockSpec((BN_ROWS, DH), lambda i: (i, 0)),
            pl.BlockSpec((BN_ROWS, DH), lambda i: (i, 0)),
            pl.BlockSpec((D, H), lambda i: (0, 0)),
            pl.BlockSpec((1, H), lambda i: (0, 0)),
            pl.BlockSpec((H, D), lambda i: (0, 0)),
            pl.BlockSpec((1, D), lambda i: (0, 0)),
        ],
        out_specs=[
            pl.BlockSpec((BN_ROWS, D), lambda i: (i, 0)),
            pl.BlockSpec((1, 2, D), lambda i: (i, 0, 0)),
        ],
        out_shape=[
            jax.ShapeDtypeStruct((N, D), jnp.float32),
            jax.ShapeDtypeStruct((NB, 2, D), jnp.float32),
        ],
        compiler_params=pltpu.CompilerParams(
            dimension_semantics=("parallel",)),
    )(zp0, zp1, W1, b12, W2, b22)


def _norm_body(z2_ref, b_ref, S_ref, g0_ref, beta0_ref, z_ref, G_ref):
    S = jnp.sum(S_ref[...], axis=0)
    m = S[0:1, :] * (1.0 / N)
    var = S[1:2, :] * (1.0 / N) - m * m
    scale = g0_ref[...] * lax.rsqrt(var + 1e-5)
    zn = jnp.maximum((z2_ref[...] - m) * scale + beta0_ref[...], 0.0)
    z_ref[...] = zn

    b = b_ref[0, 0, :]
    oh = _f32(b[:, None] == lax.broadcasted_iota(jnp.int32, (BN_ROWS, B), 1))
    G_ref[0] = lax.dot_general(oh, zn, (((0,), (0,)), ((), ())),
                               preferred_element_type=jnp.float32,
                               precision=lax.Precision.HIGHEST)


def _norm_pool(z2, batch3, S, g02, beta02):
    return pl.pallas_call(
        _norm_body,
        grid=(NB,),
        in_specs=[
            pl.BlockSpec((BN_ROWS, D), lambda i: (i, 0)),
            pl.BlockSpec((1, 1, BN_ROWS), lambda i: (i, 0, 0)),
            pl.BlockSpec((NB, 2, D), lambda i: (0, 0, 0)),
            pl.BlockSpec((1, D), lambda i: (0, 0)),
            pl.BlockSpec((1, D), lambda i: (0, 0)),
        ],
        out_specs=[
            pl.BlockSpec((BN_ROWS, D), lambda i: (i, 0)),
            pl.BlockSpec((1, B, D), lambda i: (i, 0, 0)),
        ],
        out_shape=[
            jax.ShapeDtypeStruct((N, D), jnp.float32),
            jax.ShapeDtypeStruct((NB, B, D), jnp.float32),
        ],
        compiler_params=pltpu.CompilerParams(
            dimension_semantics=("parallel",)),
    )(z2, batch3, S, g02, beta02)


def _vn_body(G_ref, vn_ref, vW1_ref, vb1_ref, vg1_ref, vbeta1_ref,
             vW2_ref, vb2_ref, vg2_ref, vbeta2_ref, out_ref):
    v = vn_ref[...] + jnp.sum(G_ref[...], axis=0)
    vh = jnp.dot(v, vW1_ref[...], preferred_element_type=jnp.float32) \
        + vb1_ref[...]
    m1 = jnp.mean(vh, axis=0, keepdims=True)
    var1 = jnp.mean((vh - m1) * (vh - m1), axis=0, keepdims=True)
    vh = jnp.maximum(vg1_ref[...] * (vh - m1) * lax.rsqrt(var1 + 1e-5)
                     + vbeta1_ref[...], 0.0)
    vh2 = jnp.dot(vh, vW2_ref[...], preferred_element_type=jnp.float32) \
        + vb2_ref[...]
    m2 = jnp.mean(vh2, axis=0, keepdims=True)
    var2 = jnp.mean((vh2 - m2) * (vh2 - m2), axis=0, keepdims=True)
    out_ref[...] = jnp.maximum(vg2_ref[...] * (vh2 - m2) * lax.rsqrt(var2 + 1e-5)
                               + vbeta2_ref[...], 0.0)


def _vn_mlp(G, vn, vW1, vb12, vg12, vbeta12, vW2, vb22, vg22, vbeta22):
    return pl.pallas_call(
        _vn_body,
        out_shape=jax.ShapeDtypeStruct((B, D), jnp.float32),
    )(G, vn, vW1, vb12, vg12, vbeta12, vW2, vb22, vg22, vbeta22)


def kernel(x, edge_index, edge_attr, batch, vn, eW, eb, W1, b1, W2, b2,
           g0, beta0, vW1, vb1, vg1, vbeta1, vW2, vb2, vg2, vbeta2):
    batch3 = batch.astype(jnp.int32).reshape(NB, 1, BN_ROWS)
    ei = edge_index.astype(jnp.int32)
    eidx = jnp.stack([ei[0].reshape(NCHUNKS, CH),
                      ei[1].reshape(NCHUNKS, CH)], axis=1)
    ea8 = edge_attr.reshape(E // 8, 8 * DE)
    W8 = jnp.kron(jnp.eye(8, dtype=jnp.float32), eW)
    eb8 = jnp.tile(eb, 8).reshape(1, 8 * D)

    h0, h1 = _h_halves(x, batch3, vn)
    ep0, ep1 = _ep_halves(ea8, W8, eb8)
    zp0, zp1 = _sc_scatter(h0, h1, ep0, ep1, eidx)
    z2, S = _mlp(zp0, zp1, W1, b1.reshape(1, H), W2, b2.reshape(1, D))
    z, G = _norm_pool(z2, batch3, S, g0.reshape(1, D), beta0.reshape(1, D))
    v_out = _vn_mlp(G, vn, vW1, vb1.reshape(1, 2 * D), vg1.reshape(1, 2 * D),
                    vbeta1.reshape(1, 2 * D), vW2, vb2.reshape(1, D),
                    vg2.reshape(1, D), vbeta2.reshape(1, D))
    return (z, v_out)


# speedup vs baseline: 1.2462x; 1.2462x over previous
import jax
import jax.numpy as jnp
from jax import lax
from jax.experimental import pallas as pl
from jax.experimental.pallas import tpu as pltpu
from jax.experimental.pallas import tpu_sc as plsc

N, E, D, H, DE, B = 10000, 160000, 256, 512, 16, 64
DH = D // 2
NB = 10
BN_ROWS = N // NB
CH = 64
NSC, NSUB = 2, 16
NCHUNKS = E // CH
CPS = NCHUNKS // NSUB
REM = NCHUNKS - CPS * NSUB
EB = 3200
NEB = E // EB


def _f32(x):
    return x.astype(jnp.float32)


def _h_body(x_ref, b_ref, vn_ref, h0_ref, h1_ref):
    b = b_ref[0, 0, :]
    oh = (b[:, None] == lax.broadcasted_iota(jnp.int32, (BN_ROWS, B), 1))
    h = x_ref[...] + jnp.dot(_f32(oh), vn_ref[...],
                             preferred_element_type=jnp.float32,
                             precision=lax.Precision.HIGHEST)
    h0_ref[...] = h[:, :DH]
    h1_ref[...] = h[:, DH:]


def _h_halves(x, batch3, vn):
    return pl.pallas_call(
        _h_body,
        grid=(NB,),
        in_specs=[
            pl.BlockSpec((BN_ROWS, D), lambda i: (i, 0)),
            pl.BlockSpec((1, 1, BN_ROWS), lambda i: (i, 0, 0)),
            pl.BlockSpec((B, D), lambda i: (0, 0)),
        ],
        out_specs=[
            pl.BlockSpec((BN_ROWS, DH), lambda i: (i, 0)),
            pl.BlockSpec((BN_ROWS, DH), lambda i: (i, 0)),
        ],
        out_shape=[
            jax.ShapeDtypeStruct((N, DH), jnp.float32),
            jax.ShapeDtypeStruct((N, DH), jnp.float32),
        ],
        compiler_params=pltpu.CompilerParams(
            dimension_semantics=("parallel",)),
    )(x, batch3, vn)


def _ep_body(eaT_ref, eW_ref, eb_ref, ep0_ref, ep1_ref):
    ep = lax.dot_general(eaT_ref[...], eW_ref[...], (((0,), (0,)), ((), ())),
                         preferred_element_type=jnp.float32) + eb_ref[...]
    ep0_ref[...] = ep[:, :DH]
    ep1_ref[...] = ep[:, DH:]


def _ep_halves(eaT, eW, eb2):
    return pl.pallas_call(
        _ep_body,
        grid=(NEB,),
        in_specs=[
            pl.BlockSpec((DE, EB), lambda i: (0, i)),
            pl.BlockSpec((DE, D), lambda i: (0, 0)),
            pl.BlockSpec((1, D), lambda i: (0, 0)),
        ],
        out_specs=[
            pl.BlockSpec((EB, DH), lambda i: (i, 0)),
            pl.BlockSpec((EB, DH), lambda i: (i, 0)),
        ],
        out_shape=[
            jax.ShapeDtypeStruct((E, DH), jnp.float32),
            jax.ShapeDtypeStruct((E, DH), jnp.float32),
        ],
        compiler_params=pltpu.CompilerParams(
            dimension_semantics=("parallel",)),
    )(eaT, eW, eb2)


def _sc_body(h0, h1, ep0, ep1, eidx, zp0, zp1,
             idx0, idx1, idx2, rows0, rows1, rows2, epv0, epv1, epv2,
             upd, sem_g, sem_e, sem_s):
    c = lax.axis_index("c")
    s = lax.axis_index("s")
    idxs = (idx0, idx1, idx2)
    rowss = (rows0, rows1, rows2)
    epvs = (epv0, epv1, epv2)
    RCH = 400
    NRCH = N // RCH

    def run_half(h_ref, ep_ref, zp_ref):
        @pl.loop(s, NRCH, step=NSUB)
        def _(rc):
            pltpu.sync_copy(h_ref.at[pl.ds(rc * RCH, RCH)],
                            upd.at[pl.ds(rc * RCH, RCH)])
        plsc.subcore_barrier()

        base = s * CPS

        def fire(slot, k):
            pltpu.sync_copy(eidx.at[k], idxs[slot])
            pltpu.async_copy(h_ref.at[idxs[slot].at[0]], rowss[slot],
                             sem_g.at[slot])
            pltpu.async_copy(ep_ref.at[pl.ds(k * CH, CH)], epvs[slot],
                             sem_e.at[slot])

        def wait_ge(slot):
            pltpu.make_async_copy(h_ref.at[idxs[slot].at[0]], rowss[slot],
                                  sem_g.at[slot]).wait()
            pltpu.make_async_copy(ep_ref.at[pl.ds(0, CH)], epvs[slot],
                                  sem_e.at[slot]).wait()

        def compute(slot):
            rows, epv = rowss[slot], epvs[slot]

            @plsc.parallel_loop(0, CH, unroll=8)
            def _(r):
                for j in range(DH // 16):
                    sl = pl.ds(j * 16, 16)
                    rows[r, sl] = jnp.maximum(rows[r, sl] + epv[r, sl], 0.0)

        def fire_scatter(slot):
            pltpu.async_copy(rowss[slot], upd.at[idxs[slot].at[1]],
                             sem_s.at[slot], add=True)

        def wait_scatter(slot):
            pltpu.make_async_copy(rowss[slot], upd.at[idxs[slot].at[1]],
                                  sem_s.at[slot]).wait()

        fire(0, base)
        fire(1, base + 1)

        @pl.loop(0, CPS // 3)
        def _(q):
            kk = 3 * q

            wait_ge(0)

            @pl.when(q > 0)
            def _():
                wait_scatter(2)
            fire(2, base + kk + 2)
            compute(0)
            fire_scatter(0)

            wait_ge(1)
            wait_scatter(0)

            @pl.when(q < CPS // 3 - 1)
            def _():
                fire(0, base + kk + 3)
            compute(1)
            fire_scatter(1)

            wait_ge(2)
            wait_scatter(1)

            @pl.when(q < CPS // 3 - 1)
            def _():
                fire(1, base + kk + 4)
            compute(2)
            fire_scatter(2)

        wait_scatter(2)

        @pl.when(s < REM)
        def _():
            k = NSUB * CPS + s
            pltpu.sync_copy(eidx.at[k], idx0)
            pltpu.async_copy(h_ref.at[idx0.at[0]], rows0, sem_g.at[0]).wait()
            pltpu.sync_copy(ep_ref.at[pl.ds(k * CH, CH)], epv0)
            compute(0)
            pltpu.sync_copy(rows0, upd.at[idx0.at[1]], add=True)

        plsc.subcore_barrier()

        @pl.loop(s, NRCH, step=NSUB)
        def _(rc):
            pltpu.sync_copy(upd.at[pl.ds(rc * RCH, RCH)],
                            zp_ref.at[pl.ds(rc * RCH, RCH)])

    @pl.when(c == 0)
    def _():
        run_half(h0, ep0, zp0)

    @pl.when(c == 1)
    def _():
        run_half(h1, ep1, zp1)


def _sc_scatter(h0, h1, ep0, ep1, eidx):
    mesh = plsc.VectorSubcoreMesh(core_axis_name="c", subcore_axis_name="s",
                                  num_cores=NSC, num_subcores=NSUB)
    k = pl.kernel(
        _sc_body,
        out_type=[
            jax.ShapeDtypeStruct((N, DH), jnp.float32),
            jax.ShapeDtypeStruct((N, DH), jnp.float32),
        ],
        mesh=mesh,
        scratch_types=[
            pltpu.VMEM((2, CH), jnp.int32),
            pltpu.VMEM((2, CH), jnp.int32),
            pltpu.VMEM((2, CH), jnp.int32),
            pltpu.VMEM((CH, DH), jnp.float32),
            pltpu.VMEM((CH, DH), jnp.float32),
            pltpu.VMEM((CH, DH), jnp.float32),
            pltpu.VMEM((CH, DH), jnp.float32),
            pltpu.VMEM((CH, DH), jnp.float32),
            pltpu.VMEM((CH, DH), jnp.float32),
            pltpu.VMEM_SHARED((N, DH), jnp.float32),
            pltpu.SemaphoreType.DMA((3,)),
            pltpu.SemaphoreType.DMA((3,)),
            pltpu.SemaphoreType.DMA((3,)),
        ],
    )
    return k(h0, h1, ep0, ep1, eidx)


def _mlp_body(zp0_ref, zp1_ref, W1_ref, b1_ref, W2_ref, b2_ref,
              z2_ref, S_ref):
    zc = jnp.concatenate([zp0_ref[...], zp1_ref[...]], axis=1)
    t = jnp.maximum(jnp.dot(zc, W1_ref[...],
                            preferred_element_type=jnp.float32)
                    + b1_ref[...], 0.0)
    z2 = jnp.maximum(jnp.dot(t, W2_ref[...],
                             preferred_element_type=jnp.float32)
                     + b2_ref[...], 0.0)
    z2_ref[...] = z2
    S_ref[0, 0, :] = jnp.sum(z2, axis=0)
    S_ref[0, 1, :] = jnp.sum(z2 * z2, axis=0)


def _mlp(zp0, zp1, W1, b12, W2, b22):
    return pl.pallas_call(
        _mlp_body,
        grid=(NB,),
        in_specs=[
            pl.BlockSpec((BN_ROWS, DH), lambda i: (i, 0)),
            pl.BlockSpec((BN_ROWS, DH), lambda i: (i, 0)),
            pl.BlockSpec((D, H), lambda i: (0, 0)),
            pl.BlockSpec((1, H), lambda i: (0, 0)),
            pl.BlockSpec((H, D), lambda i: (0, 0)),
            pl.BlockSpec((1, D), lambda i: (0, 0)),
        ],
        out_specs=[
            pl.BlockSpec((BN_ROWS, D), lambda i: (i, 0)),
            pl.BlockSpec((1, 2, D), lambda i: (i, 0, 0)),
        ],
        out_shape=[
            jax.ShapeDtypeStruct((N, D), jnp.float32),
            jax.ShapeDtypeStruct((NB, 2, D), jnp.float32),
        ],
        compiler_params=pltpu.CompilerParams(
            dimension_semantics=("parallel",)),
    )(zp0, zp1, W1, b12, W2, b22)


def _norm_body(z2_ref, b_ref, S_ref, g0_ref, beta0_ref, z_ref, G_ref):
    S = jnp.sum(S_ref[...], axis=0)
    m = S[0:1, :] * (1.0 / N)
    var = S[1:2, :] * (1.0 / N) - m * m
    scale = g0_ref[...] * lax.rsqrt(var + 1e-5)
    zn = jnp.maximum((z2_ref[...] - m) * scale + beta0_ref[...], 0.0)
    z_ref[...] = zn

    b = b_ref[0, 0, :]
    oh = _f32(b[:, None] == lax.broadcasted_iota(jnp.int32, (BN_ROWS, B), 1))
    G_ref[0] = lax.dot_general(oh, zn, (((0,), (0,)), ((), ())),
                               preferred_element_type=jnp.float32,
                               precision=lax.Precision.HIGHEST)


def _norm_pool(z2, batch3, S, g02, beta02):
    return pl.pallas_call(
        _norm_body,
        grid=(NB,),
        in_specs=[
            pl.BlockSpec((BN_ROWS, D), lambda i: (i, 0)),
            pl.BlockSpec((1, 1, BN_ROWS), lambda i: (i, 0, 0)),
            pl.BlockSpec((NB, 2, D), lambda i: (0, 0, 0)),
            pl.BlockSpec((1, D), lambda i: (0, 0)),
            pl.BlockSpec((1, D), lambda i: (0, 0)),
        ],
        out_specs=[
            pl.BlockSpec((BN_ROWS, D), lambda i: (i, 0)),
            pl.BlockSpec((1, B, D), lambda i: (i, 0, 0)),
        ],
        out_shape=[
            jax.ShapeDtypeStruct((N, D), jnp.float32),
            jax.ShapeDtypeStruct((NB, B, D), jnp.float32),
        ],
        compiler_params=pltpu.CompilerParams(
            dimension_semantics=("parallel",)),
    )(z2, batch3, S, g02, beta02)


def _vn_body(G_ref, vn_ref, vW1_ref, vb1_ref, vg1_ref, vbeta1_ref,
             vW2_ref, vb2_ref, vg2_ref, vbeta2_ref, out_ref):
    v = vn_ref[...] + jnp.sum(G_ref[...], axis=0)
    vh = jnp.dot(v, vW1_ref[...], preferred_element_type=jnp.float32) \
        + vb1_ref[...]
    m1 = jnp.mean(vh, axis=0, keepdims=True)
    var1 = jnp.mean((vh - m1) * (vh - m1), axis=0, keepdims=True)
    vh = jnp.maximum(vg1_ref[...] * (vh - m1) * lax.rsqrt(var1 + 1e-5)
                     + vbeta1_ref[...], 0.0)
    vh2 = jnp.dot(vh, vW2_ref[...], preferred_element_type=jnp.float32) \
        + vb2_ref[...]
    m2 = jnp.mean(vh2, axis=0, keepdims=True)
    var2 = jnp.mean((vh2 - m2) * (vh2 - m2), axis=0, keepdims=True)
    out_ref[...] = jnp.maximum(vg2_ref[...] * (vh2 - m2) * lax.rsqrt(var2 + 1e-5)
                               + vbeta2_ref[...], 0.0)


def _vn_mlp(G, vn, vW1, vb12, vg12, vbeta12, vW2, vb22, vg22, vbeta22):
    return pl.pallas_call(
        _vn_body,
        out_shape=jax.ShapeDtypeStruct((B, D), jnp.float32),
    )(G, vn, vW1, vb12, vg12, vbeta12, vW2, vb22, vg22, vbeta22)


def kernel(x, edge_index, edge_attr, batch, vn, eW, eb, W1, b1, W2, b2,
           g0, beta0, vW1, vb1, vg1, vbeta1, vW2, vb2, vg2, vbeta2):
    batch3 = batch.astype(jnp.int32).reshape(NB, 1, BN_ROWS)
    ei = edge_index.astype(jnp.int32)
    eidx = jnp.stack([ei[0].reshape(NCHUNKS, CH),
                      ei[1].reshape(NCHUNKS, CH)], axis=1)
    h0, h1 = _h_halves(x, batch3, vn)
    ep0, ep1 = _ep_halves(edge_attr.T, eW, eb.reshape(1, D))
    zp0, zp1 = _sc_scatter(h0, h1, ep0, ep1, eidx)
    z2, S = _mlp(zp0, zp1, W1, b1.reshape(1, H), W2, b2.reshape(1, D))
    z, G = _norm_pool(z2, batch3, S, g0.reshape(1, D), beta0.reshape(1, D))
    v_out = _vn_mlp(G, vn, vW1, vb1.reshape(1, 2 * D), vg1.reshape(1, 2 * D),
                    vbeta1.reshape(1, 2 * D), vW2, vb2.reshape(1, D),
                    vg2.reshape(1, D), vbeta2.reshape(1, D))
    return (z, v_out)
